# baseline (speedup 1.0000x reference)
import functools

import jax
import jax.numpy as jnp
from jax import lax
from jax.experimental import pallas as pl
from jax.experimental.pallas import tpu as pltpu
from jax.experimental.pallas import tpu_sc as plsc

GRID_N = 256
NPTS = 8192 * 128
NW = 32
PER_W = NPTS // NW
BC = 1024
NB = PER_W // BC
ROW = 128
NR = BC // ROW
L = 16
TABLE_W = 9 * 65536
ZERO_W = 8 * 65536
MAGIC = 12582912.0


def _pack_body(c_ref, out_ref):
    g = pl.program_id(0)
    x = c_ref[...].astype(jnp.int32)
    b = lax.broadcasted_iota(jnp.int32, (1, 32, 1, 1), 1)
    s = jnp.sum(x << b, axis=1)
    out_ref[...] = jnp.where(g < 8, s, 0)


def _pack(comb4d):
    return pl.pallas_call(
        _pack_body,
        grid=(9,),
        in_specs=[
            pl.BlockSpec((1, 32, GRID_N, GRID_N),
                         lambda g: (jnp.minimum(g, 7), 0, 0, 0)),
        ],
        out_specs=pl.BlockSpec((1, GRID_N, GRID_N), lambda g: (g, 0, 0)),
        out_shape=jax.ShapeDtypeStruct((9, GRID_N, GRID_N), jnp.int32),
    )(comb4d)


def _idx_body(p_ref, x_ref, o_ref):
    x = x_ref[0]
    y = x_ref[1]
    z = x_ref[2]
    fx = (x * p_ref[0] + p_ref[3] + MAGIC) - MAGIC
    fy = (y * p_ref[1] + p_ref[4] + MAGIC) - MAGIC
    fz = (z * p_ref[2] + p_ref[5] + MAGIC) - MAGIC
    ii = fx.astype(jnp.int32)
    jj = fy.astype(jnp.int32)
    kk = fz.astype(jnp.int32)
    oob = (ii | jj | kk) & -256
    w0 = ((ii & 224) << 11) | ((jj & 255) << 8) | (kk & 255)
    w = jnp.where(oob == 0, w0, (w0 & 65535) | ZERO_W)
    o_ref[...] = w | ((ii & 31) << 20)


def _idx(xflat3, params):
    return pl.pallas_call(
        _idx_body,
        grid=(128,),
        in_specs=[
            pl.BlockSpec(memory_space=pltpu.SMEM),
            pl.BlockSpec((3, 64, 128), lambda g: (0, g, 0)),
        ],
        out_specs=pl.BlockSpec((64, 128), lambda g: (g, 0)),
        out_shape=jax.ShapeDtypeStruct((8192, 128), jnp.int32),
    )(params, xflat3)


def _sc_body(idx_hbm, table_hbm, out_hbm, vv, wv, av, gv, ov, tsh,
             insem, gsem):
    sid = lax.axis_index("s")
    wid = sid * 2 + lax.axis_index("c")
    base_pt = wid * PER_W

    TW = TABLE_W // 16
    pltpu.sync_copy(table_hbm.at[pl.ds(sid * TW, TW)],
                    tsh.at[pl.ds(sid * TW, TW)])
    plsc.subcore_barrier()

    pltpu.async_copy(idx_hbm.at[pl.ds(base_pt, BC)], vv, insem)

    def block_body(t, carry):
        pt0 = base_pt + t * BC
        pltpu.make_async_copy(idx_hbm.at[pl.ds(0, BC)], vv, insem).wait()

        for r in range(NR):
            for gg in range(ROW // L):
                off = r * ROW + gg * L
                v = vv[pl.ds(off, L)]
                wv[pl.ds(off, L)] = v & 1048575
                av[pl.ds(off, L)] = lax.shift_right_logical(v, 20)
            pltpu.async_copy(tsh.at[wv.at[pl.ds(r * ROW, ROW)]],
                             gv.at[pl.ds(r * ROW, ROW)], gsem)

        ptn = base_pt + lax.rem(t + 1, NB) * BC
        pltpu.async_copy(idx_hbm.at[pl.ds(ptn, BC)], vv, insem)

        pltpu.make_async_copy(table_hbm.at[pl.ds(0, BC)], gv, gsem).wait()

        for g in range(BC // L):
            w = gv[pl.ds(g * L, L)]
            a = av[pl.ds(g * L, L)]
            ov[pl.ds(g * L, L)] = lax.shift_right_logical(w, a) & 1

        pltpu.sync_copy(ov, out_hbm.at[pl.ds(pt0, BC)])
        return carry

    lax.fori_loop(0, NB, block_body, 0)
    pltpu.make_async_copy(idx_hbm.at[pl.ds(0, BC)], vv, insem).wait()


@functools.partial(
    pl.kernel,
    out_type=jax.ShapeDtypeStruct((NPTS,), jnp.int32),
    mesh=plsc.VectorSubcoreMesh(core_axis_name="c", subcore_axis_name="s"),
    compiler_params=pltpu.CompilerParams(needs_layout_passes=False),
    scratch_types=[
        pltpu.VMEM((BC,), jnp.int32),
        pltpu.VMEM((BC,), jnp.int32),
        pltpu.VMEM((BC,), jnp.int32),
        pltpu.VMEM((BC,), jnp.int32),
        pltpu.VMEM((BC,), jnp.int32),
        pltpu.VMEM_SHARED((TABLE_W,), jnp.int32),
        pltpu.SemaphoreType.DMA,
        pltpu.SemaphoreType.DMA,
    ],
)
def _lookup(idx_hbm, table_hbm, out_hbm, *scratch):
    _sc_body(idx_hbm, table_hbm, out_hbm, *scratch)


def kernel(xyz, mask, bound_mask, xyz2ijk_scale, xyz2ijk_shift):
    shape = xyz.shape[:-1]
    comb_u8 = jnp.logical_and(mask, bound_mask).astype(jnp.uint8)
    packed = _pack(comb_u8.reshape(8, 32, GRID_N, GRID_N)).reshape(-1)
    xflat3 = jnp.transpose(xyz, (2, 0, 1))
    params = jnp.concatenate([
        xyz2ijk_scale.astype(jnp.float32),
        xyz2ijk_shift.astype(jnp.float32),
    ])
    idx = _idx(xflat3, params).reshape(-1)
    flat = _lookup(idx, packed)
    return flat.astype(jnp.bool_).reshape(shape)


# speedup vs baseline: 1.5411x; 1.5411x over previous
import functools

import jax
import jax.numpy as jnp
from jax import lax
from jax.experimental import pallas as pl
from jax.experimental.pallas import tpu as pltpu
from jax.experimental.pallas import tpu_sc as plsc

GRID_N = 256
NPTS = 8192 * 128
NW = 32
PER_W = NPTS // NW
BC = 1024
NB = PER_W // BC
ROW = 128
NR = BC // ROW
L = 16
TABLE_W = 9 * 65536
ZERO_W = 8 * 65536
MAGIC = 12582912.0


def _pack_body(c_ref, out_ref):
    g = pl.program_id(0)
    x = c_ref[...].astype(jnp.int32)
    b = lax.broadcasted_iota(jnp.int32, (1, 32, 1, 1), 1)
    s = jnp.sum(x << b, axis=1)
    out_ref[...] = jnp.where(g < 8, s, 0)


def _pack(comb4d):
    return pl.pallas_call(
        _pack_body,
        grid=(9,),
        in_specs=[
            pl.BlockSpec((1, 32, GRID_N, GRID_N),
                         lambda g: (jnp.minimum(g, 7), 0, 0, 0)),
        ],
        out_specs=pl.BlockSpec((1, GRID_N, GRID_N), lambda g: (g, 0, 0)),
        out_shape=jax.ShapeDtypeStruct((9, GRID_N, GRID_N), jnp.int32),
    )(comb4d)


def _idx_body(p_ref, x_ref, o_ref):
    x = x_ref[0]
    y = x_ref[1]
    z = x_ref[2]
    fx = (x * p_ref[0] + p_ref[3] + MAGIC) - MAGIC
    fy = (y * p_ref[1] + p_ref[4] + MAGIC) - MAGIC
    fz = (z * p_ref[2] + p_ref[5] + MAGIC) - MAGIC
    ii = fx.astype(jnp.int32)
    jj = fy.astype(jnp.int32)
    kk = fz.astype(jnp.int32)
    oob = (ii | jj | kk) & -256
    w0 = ((ii & 224) << 11) | ((jj & 255) << 8) | (kk & 255)
    w = jnp.where(oob == 0, w0, (w0 & 65535) | ZERO_W)
    o_ref[...] = w | ((ii & 31) << 20)


def _idx(xflat3, params):
    return pl.pallas_call(
        _idx_body,
        grid=(16,),
        in_specs=[
            pl.BlockSpec(memory_space=pltpu.SMEM),
            pl.BlockSpec((3, 512, 128), lambda g: (0, g, 0)),
        ],
        out_specs=pl.BlockSpec((512, 128), lambda g: (g, 0)),
        out_shape=jax.ShapeDtypeStruct((8192, 128), jnp.int32),
    )(params, xflat3)


def _sc_body(idx_hbm, table_hbm, out_hbm, vv, wv, av, gv, ov, tsh,
             insem, gsem):
    sid = lax.axis_index("s")
    wid = sid * 2 + lax.axis_index("c")
    base_pt = wid * PER_W

    TW = TABLE_W // 16
    pltpu.sync_copy(table_hbm.at[pl.ds(sid * TW, TW)],
                    tsh.at[pl.ds(sid * TW, TW)])
    plsc.subcore_barrier()

    pltpu.async_copy(idx_hbm.at[pl.ds(base_pt, BC)], vv, insem)

    def block_body(t, carry):
        pt0 = base_pt + t * BC
        pltpu.make_async_copy(idx_hbm.at[pl.ds(0, BC)], vv, insem).wait()

        for r in range(NR):
            for gg in range(ROW // L):
                off = r * ROW + gg * L
                v = vv[pl.ds(off, L)]
                wv[pl.ds(off, L)] = v & 1048575
                av[pl.ds(off, L)] = lax.shift_right_logical(v, 20)
            pltpu.async_copy(tsh.at[wv.at[pl.ds(r * ROW, ROW)]],
                             gv.at[pl.ds(r * ROW, ROW)], gsem)

        ptn = base_pt + lax.rem(t + 1, NB) * BC
        pltpu.async_copy(idx_hbm.at[pl.ds(ptn, BC)], vv, insem)

        pltpu.make_async_copy(table_hbm.at[pl.ds(0, BC)], gv, gsem).wait()

        for g in range(BC // L):
            w = gv[pl.ds(g * L, L)]
            a = av[pl.ds(g * L, L)]
            ov[pl.ds(g * L, L)] = lax.shift_right_logical(w, a) & 1

        pltpu.sync_copy(ov, out_hbm.at[pl.ds(pt0, BC)])
        return carry

    lax.fori_loop(0, NB, block_body, 0)
    pltpu.make_async_copy(idx_hbm.at[pl.ds(0, BC)], vv, insem).wait()


@functools.partial(
    pl.kernel,
    out_type=jax.ShapeDtypeStruct((NPTS,), jnp.int32),
    mesh=plsc.VectorSubcoreMesh(core_axis_name="c", subcore_axis_name="s"),
    compiler_params=pltpu.CompilerParams(needs_layout_passes=False),
    scratch_types=[
        pltpu.VMEM((BC,), jnp.int32),
        pltpu.VMEM((BC,), jnp.int32),
        pltpu.VMEM((BC,), jnp.int32),
        pltpu.VMEM((BC,), jnp.int32),
        pltpu.VMEM((BC,), jnp.int32),
        pltpu.VMEM_SHARED((TABLE_W,), jnp.int32),
        pltpu.SemaphoreType.DMA,
        pltpu.SemaphoreType.DMA,
    ],
)
def _lookup(idx_hbm, table_hbm, out_hbm, *scratch):
    _sc_body(idx_hbm, table_hbm, out_hbm, *scratch)


def kernel(xyz, mask, bound_mask, xyz2ijk_scale, xyz2ijk_shift):
    shape = xyz.shape[:-1]
    comb_u8 = jnp.logical_and(mask, bound_mask).astype(jnp.uint8)
    packed = _pack(comb_u8.reshape(8, 32, GRID_N, GRID_N)).reshape(-1)
    xflat3 = jnp.transpose(xyz, (2, 0, 1))
    params = jnp.concatenate([
        xyz2ijk_scale.astype(jnp.float32),
        xyz2ijk_shift.astype(jnp.float32),
    ])
    idx = _idx(xflat3, params).reshape(-1)
    flat = _lookup(idx, packed)
    return flat.astype(jnp.bool_).reshape(shape)


# speedup vs baseline: 1.6083x; 1.0436x over previous
import functools

import jax
import jax.numpy as jnp
from jax import lax
from jax.experimental import pallas as pl
from jax.experimental.pallas import tpu as pltpu
from jax.experimental.pallas import tpu_sc as plsc

GRID_N = 256
NPTS = 8192 * 128
NW = 32
PER_W = NPTS // NW
BC = 1024
NB = PER_W // BC
ROW = 128
NR = BC // ROW
L = 16
TABLE_W = 9 * 65536
ZERO_W = 8 * 65536
MAGIC = 12582912.0


def _pack_body(c_ref, out_ref):
    g = pl.program_id(0)
    x = c_ref[...].astype(jnp.int32)
    b = lax.broadcasted_iota(jnp.int32, (1, 32, 1, 1), 1)
    s = jnp.sum(x << b, axis=1)
    out_ref[...] = jnp.where(g < 8, s, 0)


def _pack(comb4d):
    return pl.pallas_call(
        _pack_body,
        grid=(9,),
        in_specs=[
            pl.BlockSpec((1, 32, GRID_N, GRID_N),
                         lambda g: (jnp.minimum(g, 7), 0, 0, 0)),
        ],
        out_specs=pl.BlockSpec((1, GRID_N, GRID_N), lambda g: (g, 0, 0)),
        out_shape=jax.ShapeDtypeStruct((9, GRID_N, GRID_N), jnp.int32),
    )(comb4d)


def _idx_body(p_ref, x_ref, o_ref):
    x = x_ref[0]
    y = x_ref[1]
    z = x_ref[2]
    fx = (x * p_ref[0] + p_ref[3] + MAGIC) - MAGIC
    fy = (y * p_ref[1] + p_ref[4] + MAGIC) - MAGIC
    fz = (z * p_ref[2] + p_ref[5] + MAGIC) - MAGIC
    ii = fx.astype(jnp.int32)
    jj = fy.astype(jnp.int32)
    kk = fz.astype(jnp.int32)
    oob = (ii | jj | kk) & -256
    w0 = ((ii & 224) << 11) | ((jj & 255) << 8) | (kk & 255)
    w = jnp.where(oob == 0, w0, (w0 & 65535) | ZERO_W)
    o_ref[...] = w | ((ii & 31) << 20)


def _idx(xflat3, params):
    return pl.pallas_call(
        _idx_body,
        grid=(8,),
        in_specs=[
            pl.BlockSpec(memory_space=pltpu.SMEM),
            pl.BlockSpec((3, 1024, 128), lambda g: (0, g, 0)),
        ],
        out_specs=pl.BlockSpec((1024, 128), lambda g: (g, 0)),
        out_shape=jax.ShapeDtypeStruct((8192, 128), jnp.int32),
    )(params, xflat3)


def _sc_body(idx_hbm, table_hbm, out_hbm, vv, wv, av, gv, ov, tsh,
             insem, gsem):
    sid = lax.axis_index("s")
    wid = sid * 2 + lax.axis_index("c")
    base_pt = wid * PER_W

    TW = TABLE_W // 16
    pltpu.sync_copy(table_hbm.at[pl.ds(sid * TW, TW)],
                    tsh.at[pl.ds(sid * TW, TW)])
    plsc.subcore_barrier()

    pltpu.async_copy(idx_hbm.at[pl.ds(base_pt, BC)], vv, insem)

    def block_body(t, carry):
        pt0 = base_pt + t * BC
        pltpu.make_async_copy(idx_hbm.at[pl.ds(0, BC)], vv, insem).wait()

        for r in range(NR):
            for gg in range(ROW // L):
                off = r * ROW + gg * L
                v = vv[pl.ds(off, L)]
                wv[pl.ds(off, L)] = v & 1048575
                av[pl.ds(off, L)] = lax.shift_right_logical(v, 20)
            pltpu.async_copy(tsh.at[wv.at[pl.ds(r * ROW, ROW)]],
                             gv.at[pl.ds(r * ROW, ROW)], gsem)

        ptn = base_pt + lax.rem(t + 1, NB) * BC
        pltpu.async_copy(idx_hbm.at[pl.ds(ptn, BC)], vv, insem)

        pltpu.make_async_copy(table_hbm.at[pl.ds(0, BC)], gv, gsem).wait()

        for g in range(BC // L):
            w = gv[pl.ds(g * L, L)]
            a = av[pl.ds(g * L, L)]
            ov[pl.ds(g * L, L)] = lax.shift_right_logical(w, a) & 1

        pltpu.sync_copy(ov, out_hbm.at[pl.ds(pt0, BC)])
        return carry

    lax.fori_loop(0, NB, block_body, 0)
    pltpu.make_async_copy(idx_hbm.at[pl.ds(0, BC)], vv, insem).wait()


@functools.partial(
    pl.kernel,
    out_type=jax.ShapeDtypeStruct((NPTS,), jnp.int32),
    mesh=plsc.VectorSubcoreMesh(core_axis_name="c", subcore_axis_name="s"),
    compiler_params=pltpu.CompilerParams(needs_layout_passes=False),
    scratch_types=[
        pltpu.VMEM((BC,), jnp.int32),
        pltpu.VMEM((BC,), jnp.int32),
        pltpu.VMEM((BC,), jnp.int32),
        pltpu.VMEM((BC,), jnp.int32),
        pltpu.VMEM((BC,), jnp.int32),
        pltpu.VMEM_SHARED((TABLE_W,), jnp.int32),
        pltpu.SemaphoreType.DMA,
        pltpu.SemaphoreType.DMA,
    ],
)
def _lookup(idx_hbm, table_hbm, out_hbm, *scratch):
    _sc_body(idx_hbm, table_hbm, out_hbm, *scratch)


def kernel(xyz, mask, bound_mask, xyz2ijk_scale, xyz2ijk_shift):
    shape = xyz.shape[:-1]
    comb_u8 = jnp.logical_and(mask, bound_mask).astype(jnp.uint8)
    packed = _pack(comb_u8.reshape(8, 32, GRID_N, GRID_N)).reshape(-1)
    xflat3 = jnp.transpose(xyz, (2, 0, 1))
    params = jnp.concatenate([
        xyz2ijk_scale.astype(jnp.float32),
        xyz2ijk_shift.astype(jnp.float32),
    ])
    idx = _idx(xflat3, params).reshape(-1)
    flat = _lookup(idx, packed)
    return flat.astype(jnp.bool_).reshape(shape)
